# baseline (speedup 1.0000x reference)
import functools

import jax
import jax.numpy as jnp
from jax import lax
from jax.experimental import pallas as pl
from jax.experimental.pallas import tpu as pltpu
from jax.experimental.pallas import tpu_sc as plsc

DEPTH = 24
VQT_NUM = 10
EMBED_DIM = 1024
BATCH = 256

_info = plsc.get_sparse_core_info()
_NC = _info.num_cores
_NS = _info.num_subcores
_NL = _info.num_lanes
_NW = _NC * _NS
_B_PER_W = BATCH // _NW

_mesh = plsc.VectorSubcoreMesh(core_axis_name="c", subcore_axis_name="s")


@functools.partial(
    pl.kernel,
    mesh=_mesh,
    out_type=jax.ShapeDtypeStruct((VQT_NUM, BATCH, EMBED_DIM), jnp.float32),
    scratch_types=[
        pltpu.VMEM((_NL,), jnp.int32),
        pltpu.VMEM((VQT_NUM, EMBED_DIM), jnp.float32),
        pltpu.SemaphoreType.DMA,
        pltpu.SemaphoreType.DMA,
    ],
)
def _vqt_expand(table_hbm, idx_hbm, out_hbm, idx_v, row_v, gsem, wsem):
    wid = lax.axis_index("s") * _NC + lax.axis_index("c")
    base = wid * _B_PER_W
    pltpu.sync_copy(idx_hbm, idx_v)
    layer = idx_v[...][0]
    gathers = [
        pltpu.async_copy(table_hbm.at[v].at[layer], row_v.at[v], gsem)
        for v in range(VQT_NUM)
    ]
    for g in gathers:
        g.wait()
    copies = [
        pltpu.async_copy(row_v.at[v], out_hbm.at[v].at[base + j], wsem)
        for v in range(VQT_NUM)
        for j in range(_B_PER_W)
    ]
    for c in copies:
        c.wait()


def kernel(query_prompt_embeddings, index, batch_size):
    del batch_size
    table_t = jnp.transpose(query_prompt_embeddings, (1, 0, 2))
    idx = jnp.zeros((_NL,), jnp.int32).at[0].set(index)
    out = _vqt_expand(table_t, idx)
    return jnp.transpose(out, (1, 0, 2))


# speedup vs baseline: 1.0877x; 1.0877x over previous
import functools

import jax
import jax.numpy as jnp
from jax import lax
from jax.experimental import pallas as pl
from jax.experimental.pallas import tpu as pltpu
from jax.experimental.pallas import tpu_sc as plsc

DEPTH = 24
VQT_NUM = 10
EMBED_DIM = 1024
BATCH = 256

_info = plsc.get_sparse_core_info()
_NC = _info.num_cores
_NS = _info.num_subcores
_NL = _info.num_lanes
_NW = _NC * _NS
_B_PER_W = BATCH // _NW

_mesh = plsc.VectorSubcoreMesh(core_axis_name="c", subcore_axis_name="s")


@functools.partial(
    pl.kernel,
    mesh=_mesh,
    out_type=jax.ShapeDtypeStruct((VQT_NUM, BATCH, EMBED_DIM), jnp.float32),
    scratch_types=[
        pltpu.VMEM((_NL,), jnp.int32),
        pltpu.VMEM((VQT_NUM, EMBED_DIM), jnp.float32),
        pltpu.SemaphoreType.DMA,
        pltpu.SemaphoreType.DMA,
    ],
)
def _vqt_expand(table_hbm, idx_hbm, out_hbm, idx_v, row_v, gsem, wsem):
    wid = lax.axis_index("s") * _NC + lax.axis_index("c")
    base = wid * _B_PER_W
    pltpu.sync_copy(idx_hbm, idx_v)
    layer = idx_v[...][0]
    gathers = [
        pltpu.async_copy(table_hbm.at[v].at[layer], row_v.at[v], gsem)
        for v in range(VQT_NUM)
    ]
    for g in gathers:
        g.wait()

    def _fire(v, carry):
        for j in range(_B_PER_W):
            pltpu.async_copy(row_v.at[v], out_hbm.at[v].at[base + j], wsem)
        return carry

    lax.fori_loop(0, VQT_NUM, _fire, 0, unroll=False)

    def _drain(v, carry):
        for j in range(_B_PER_W):
            pltpu.make_async_copy(
                row_v.at[v], out_hbm.at[v].at[base + j], wsem
            ).wait()
        return carry

    lax.fori_loop(0, VQT_NUM, _drain, 0, unroll=False)


def kernel(query_prompt_embeddings, index, batch_size):
    del batch_size
    table_t = jnp.transpose(query_prompt_embeddings, (1, 0, 2))
    idx = jnp.zeros((_NL,), jnp.int32).at[0].set(index)
    out = _vqt_expand(table_t, idx)
    return jnp.transpose(out, (1, 0, 2))


# speedup vs baseline: 1.1146x; 1.0247x over previous
import functools

import jax
import jax.numpy as jnp
from jax import lax
from jax.experimental import pallas as pl
from jax.experimental.pallas import tpu as pltpu
from jax.experimental.pallas import tpu_sc as plsc

DEPTH = 24
VQT_NUM = 10
EMBED_DIM = 1024
BATCH = 256

_info = plsc.get_sparse_core_info()
_NC = _info.num_cores
_NS = _info.num_subcores
_NL = _info.num_lanes
_NW = _NC * _NS
_B_PER_W = BATCH // _NW

_mesh = plsc.VectorSubcoreMesh(core_axis_name="c", subcore_axis_name="s")


@functools.partial(
    pl.kernel,
    mesh=_mesh,
    out_type=jax.ShapeDtypeStruct((VQT_NUM, BATCH, EMBED_DIM), jnp.float32),
    scratch_types=[
        pltpu.VMEM((_NL,), jnp.int32),
        pltpu.VMEM((VQT_NUM, 1, EMBED_DIM), jnp.float32),
        pltpu.SemaphoreType.DMA,
        pltpu.SemaphoreType.DMA,
    ],
)
def _vqt_expand(table_hbm, idx_hbm, out_hbm, idx_v, row_v, gsem, wsem):
    wid = lax.axis_index("s") * _NC + lax.axis_index("c")
    base = wid * _B_PER_W
    pltpu.sync_copy(idx_hbm, idx_v)
    layer = idx_v[...][0]
    pltpu.async_copy(
        table_hbm.at[:, pl.ds(layer, 1), :], row_v, gsem
    ).wait()
    copies = [
        pltpu.async_copy(
            row_v, out_hbm.at[:, pl.ds(base + j, 1), :], wsem
        )
        for j in range(_B_PER_W)
    ]
    for c in copies:
        c.wait()


def kernel(query_prompt_embeddings, index, batch_size):
    del batch_size
    table_t = jnp.transpose(query_prompt_embeddings, (1, 0, 2))
    idx = jnp.zeros((_NL,), jnp.int32).at[0].set(index)
    out = _vqt_expand(table_t, idx)
    return jnp.transpose(out, (1, 0, 2))
